# baseline (speedup 1.0000x reference)
import jax
import jax.numpy as jnp
from jax import lax
from jax.experimental import pallas as pl
from jax.experimental.pallas import tpu as pltpu
from jax.experimental.pallas import tpu_sc as plsc

N_POINTS = 160000
N_FRAMES = 10000
N_SEQS = 128
FEAT = 256
NUM_SB = 8
LANE = 16
NFG = FEAT // LANE
NC = 2
NS = 16
NW = NC * NS


def _lenoff(opw: int) -> int:
  return ((opw + 32) // 16) * 16


def _seg_max_sc(n_items: int, n_out_pad: int, opw: int, ch: int):
  assert n_out_pad == NW * opw
  lenoff = _lenoff(opw)

  def body(x_hbm, offs_hbm, out_hbm, xbuf, offbuf, outbuf):
    w = lax.axis_index("s") * NC + lax.axis_index("c")
    fbase = w * opw
    abase = pl.multiple_of((fbase // 16) * 16, 16)
    pltpu.sync_copy(offs_hbm.at[pl.ds(abase, lenoff)], offbuf)
    lane = lax.iota(jnp.int32, 16)
    minus1 = jnp.full((16,), -1, jnp.int32)
    ninf = jnp.full((16,), -jnp.inf, jnp.float32)

    def off_at(i_rel):
      g = pl.multiple_of((i_rel // 16) * 16, 16)
      vv = offbuf[pl.ds(g, 16)]
      return jnp.max(jnp.where(lane == (i_rel % 16), vv, minus1))

    start = off_at(fbase - abase)
    end = off_at(fbase + opw - abase)
    astart = (start // 16) * 16
    nch = jnp.maximum((end - astart + ch - 1) // ch, 0)

    def init_body(i, _):
      for f in range(NFG):
        outbuf[i, pl.ds(f * 16, 16)] = ninf
      return 0
    lax.fori_loop(0, opw, init_body, 0)

    def chunk_body(c, carry):
      r0, acc0 = carry
      lo = astart + c * ch
      cbase = pl.multiple_of((jnp.minimum(lo, n_items - ch) // 16) * 16, 16)
      pltpu.sync_copy(x_hbm.at[pl.ds(cbase, ch)], xbuf)
      s_r = off_at(r0 - abase)
      j0 = jnp.clip(s_r - cbase, 0, ch)

      def cond(st):
        r_, j_, _ = st
        return jnp.logical_and(j_ < ch, r_ < fbase + opw)

      def run_body(st):
        r_, j_, acc_ = st
        e_g = off_at(r_ + 1 - abase)
        e_l = jnp.clip(e_g - cbase, 0, ch)

        def pb(p, a):
          return tuple(jnp.maximum(a[f], xbuf[p, pl.ds(f * 16, 16)])
                       for f in range(NFG))

        acc_ = lax.fori_loop(j_, e_l, pb, acc_)
        donef = e_l < ch

        @pl.when(donef)
        def _():
          row = r_ - fbase
          for f in range(NFG):
            outbuf[row, pl.ds(f * 16, 16)] = acc_[f]

        acc_ = tuple(jnp.where(donef, ninf, a) for a in acc_)
        r_ = jnp.where(donef, r_ + 1, r_)
        return (r_, e_l, acc_)

      r0, _, acc0 = lax.while_loop(cond, run_body, (r0, j0, acc0))
      return r0, acc0

    r_fin, acc_fin = lax.fori_loop(0, nch, chunk_body,
                                   (fbase, (ninf,) * NFG))

    @pl.when(r_fin < fbase + opw)
    def _():
      row = r_fin - fbase
      for f in range(NFG):
        outbuf[row, pl.ds(f * 16, 16)] = acc_fin[f]

    pltpu.sync_copy(outbuf, out_hbm.at[pl.ds(fbase, opw)])

  mesh = plsc.VectorSubcoreMesh(core_axis_name="c", subcore_axis_name="s")
  return pl.kernel(
      body,
      out_type=jax.ShapeDtypeStruct((n_out_pad, FEAT), jnp.float32),
      mesh=mesh,
      scratch_types=[
          pltpu.VMEM((ch, FEAT), jnp.float32),
          pltpu.VMEM((lenoff,), jnp.int32),
          pltpu.VMEM((opw, FEAT), jnp.float32),
      ],
      compiler_params=pltpu.CompilerParams(needs_layout_passes=False),
  )


def _heads_tc(fp_ref, sp_ref, bbc_ref, cosr_ref, sinr_ref, swap_ref,
              Wc_ref, bc_ref, Wv_ref, bv_ref, Wy_ref, by_ref,
              Wsb_ref, bsb_ref, Wsr_ref, bsr_ref,
              cen_ref, vel_ref, yaw_ref, sr_ref, sb_ref):
  fp = fp_ref[...]
  hp = jax.lax.Precision.HIGHEST
  cen_ref[...] = (jnp.dot(fp, Wc_ref[...], precision=hp,
                          preferred_element_type=jnp.float32)
                  + bc_ref[...] + bbc_ref[...])
  vel_ref[...] = (jnp.dot(fp, Wv_ref[...], precision=hp,
                          preferred_element_type=jnp.float32) + bv_ref[...])
  y = (jnp.dot(fp, Wy_ref[...], precision=hp,
               preferred_element_type=jnp.float32) + by_ref[...])
  ya = jnp.dot(y, swap_ref[...], precision=hp,
               preferred_element_type=jnp.float32)
  yaw_ref[...] = cosr_ref[...] * y + sinr_ref[...] * ya

  @pl.when(pl.program_id(0) == 0)
  def _():
    sp = sp_ref[...]
    sr_ref[...] = (jnp.dot(sp, Wsr_ref[...], precision=hp,
                           preferred_element_type=jnp.float32) + bsr_ref[...])
    logits = (jnp.dot(sp, Wsb_ref[...], precision=hp,
                      preferred_element_type=jnp.float32) + bsb_ref[...])
    m = jnp.max(logits, axis=1, keepdims=True)
    e = jnp.exp(logits - m)
    sb_ref[...] = e / jnp.sum(e, axis=1, keepdims=True)


def kernel(x, bbox_center, bbox_cos_yaw, bbox_sin_yaw, raw_xyz,
           point2frameidx, frame2batchidx,
           Wc, bc, Wy, by, Wv, bv, Wsb, bsb, Wsr, bsr):
  del raw_xyz
  opw1 = 320
  nfp = NW * opw1
  opw2 = N_SEQS // NW

  len1 = ((NW - 1) * opw1 // 16) * 16 + _lenoff(opw1)
  offs1 = jnp.searchsorted(
      point2frameidx, jnp.arange(len1, dtype=jnp.int32)).astype(jnp.int32)
  len2 = ((NW - 1) * opw2 // 16) * 16 + _lenoff(opw2)
  offs2 = jnp.searchsorted(
      frame2batchidx, jnp.arange(len2, dtype=jnp.int32)).astype(jnp.int32)

  seg1 = _seg_max_sc(N_POINTS, nfp, opw1, 64)
  fp = seg1(x, offs1)

  seg2 = _seg_max_sc(N_FRAMES, N_SEQS, opw2, 64)
  sp = seg2(fp, offs2)

  npad = nfp - N_FRAMES
  bbc_p = jnp.pad(bbox_center, ((0, npad), (0, 0)))
  cos_p = jnp.pad(bbox_cos_yaw.reshape(-1, 1), ((0, npad), (0, 0)))
  sin_p = jnp.pad(bbox_sin_yaw.reshape(-1, 1), ((0, npad), (0, 0)))

  swap = jnp.array([[0.0, -1.0], [1.0, 0.0]], dtype=jnp.float32)
  fb = 1024
  ng = nfp // fb
  row_blk = lambda r: pl.BlockSpec((fb, r), lambda i: (i, 0))
  rep = lambda a, b: pl.BlockSpec((a, b), lambda i: (0, 0))
  outs = pl.pallas_call(
      _heads_tc,
      grid=(ng,),
      in_specs=[
          row_blk(FEAT), rep(N_SEQS, FEAT), row_blk(3), row_blk(1), row_blk(1),
          rep(2, 2),
          rep(FEAT, 3), rep(1, 3), rep(FEAT, 3), rep(1, 3),
          rep(FEAT, 2), rep(1, 2),
          rep(FEAT, NUM_SB), rep(1, NUM_SB),
          rep(FEAT, NUM_SB * 3), rep(1, NUM_SB * 3),
      ],
      out_specs=[
          row_blk(3), row_blk(3), row_blk(2),
          rep(N_SEQS, NUM_SB * 3), rep(N_SEQS, NUM_SB),
      ],
      out_shape=[
          jax.ShapeDtypeStruct((nfp, 3), jnp.float32),
          jax.ShapeDtypeStruct((nfp, 3), jnp.float32),
          jax.ShapeDtypeStruct((nfp, 2), jnp.float32),
          jax.ShapeDtypeStruct((N_SEQS, NUM_SB * 3), jnp.float32),
          jax.ShapeDtypeStruct((N_SEQS, NUM_SB), jnp.float32),
      ],
  )(fp, sp, bbc_p, cos_p, sin_p, swap,
    Wc, bc.reshape(1, -1), Wv, bv.reshape(1, -1), Wy, by.reshape(1, -1),
    Wsb, bsb.reshape(1, -1), Wsr, bsr.reshape(1, -1))
  centers, velocities, yaw_sincos, size_residual, size_bin = outs
  return (centers[:N_FRAMES], velocities[:N_FRAMES], yaw_sincos[:N_FRAMES],
          size_residual, size_bin)
